# baseline (speedup 1.0000x reference)
import jax
import jax.numpy as jnp
from jax.experimental import pallas as pl
from jax.experimental.pallas import tpu as pltpu

_BR = 128
_BC = 128
_DEPTH = 8
_OD = 6


def _body(nrows_ref, ncols_ref, x_hbm, o_hbm, xbuf, obuf, sems, osems):
    c = o_hbm.shape[2]
    n_rb = x_hbm.shape[1] // _BR
    n_cb = c // _BC
    num_steps = pl.num_programs(0)
    g = pl.program_id(0)
    s = jax.lax.rem(g, _DEPTH)
    so = jax.lax.rem(g, _OD)

    def chunk_copy(gi, slot, k):
        b = gi // n_rb
        rb = jax.lax.rem(gi, n_rb)
        row0 = rb * _BR
        return pltpu.make_async_copy(
            x_hbm.at[b, pl.ds(row0, _BR), pl.ds(k * _BC, _BC)],
            xbuf.at[slot, :, pl.ds(k * _BC, _BC)],
            sems.at[slot, k],
        )

    def live_chunk(gi, k):
        b = gi // n_rb
        rb = jax.lax.rem(gi, n_rb)
        return (rb * _BR < nrows_ref[b]) & (k * _BC < ncols_ref[b])

    def start_copies(gi, slot):
        for k in range(n_cb):
            @pl.when(live_chunk(gi, k))
            def _():
                chunk_copy(gi, slot, k).start()

    def wait_copies(gi, slot):
        for k in range(n_cb):
            @pl.when(live_chunk(gi, k))
            def _():
                chunk_copy(gi, slot, k).wait()

    def out_copy(gi, slot):
        b = gi // n_rb
        rb = jax.lax.rem(gi, n_rb)
        return pltpu.make_async_copy(
            obuf.at[slot],
            o_hbm.at[b, pl.ds(rb * _BR, _BR), :],
            osems.at[slot],
        )

    @pl.when(g == 0)
    def _():
        for i in range(_DEPTH - 1):
            start_copies(i, i)

    @pl.when(g + (_DEPTH - 1) < num_steps)
    def _():
        start_copies(g + (_DEPTH - 1), jax.lax.rem(g + (_DEPTH - 1), _DEPTH))

    @pl.when(g >= _OD)
    def _():
        out_copy(g - _OD, so).wait()

    wait_copies(g, s)

    b = g // n_rb
    rb = jax.lax.rem(g, n_rb)
    nr = nrows_ref[b]
    nc = ncols_ref[b]
    row0 = rb * _BR

    @pl.when(row0 >= nr)
    def _():
        obuf[so] = jnp.zeros((_BR, c), jnp.float32)

    @pl.when(row0 < nr)
    def _():
        x = xbuf[s]
        colmask = jax.lax.broadcasted_iota(jnp.int32, (_BR, c), 1) < nc
        masked = jnp.where(colmask, x, -jnp.inf)
        m = jnp.max(masked, axis=1, keepdims=True)
        safe_m = jnp.where(jnp.isfinite(m), m, 0.0)
        e = jnp.exp(masked - safe_m)
        denom = jnp.sum(e, axis=1, keepdims=True)
        rowvalid = (row0 + jax.lax.broadcasted_iota(jnp.int32, (_BR, 1), 0)) < nr
        inv = jnp.where(rowvalid & (denom > 0),
                        1.0 / jnp.maximum(denom, 1e-30), 0.0)
        obuf[so] = e * inv

    out_copy(g, so).start()

    @pl.when(g == num_steps - 1)
    def _():
        for i in range(_OD):
            out_copy(g - i, jax.lax.rem(g - i, _OD)).wait()


def kernel(sim_mat, nrows, ncols):
    bsz, r, c = sim_mat.shape
    n_rb = r // _BR

    grid_spec = pltpu.PrefetchScalarGridSpec(
        num_scalar_prefetch=2,
        grid=(bsz * n_rb,),
        in_specs=[pl.BlockSpec(memory_space=pltpu.MemorySpace.HBM)],
        out_specs=pl.BlockSpec(memory_space=pltpu.MemorySpace.HBM),
        scratch_shapes=[
            pltpu.VMEM((_DEPTH, _BR, c), jnp.float32),
            pltpu.VMEM((_OD, _BR, c), jnp.float32),
            pltpu.SemaphoreType.DMA((_DEPTH, c // _BC)),
            pltpu.SemaphoreType.DMA((_OD,)),
        ],
    )
    return pl.pallas_call(
        _body,
        grid_spec=grid_spec,
        out_shape=jax.ShapeDtypeStruct((bsz, r, c), sim_mat.dtype),
    )(nrows.astype(jnp.int32), ncols.astype(jnp.int32), sim_mat)


# speedup vs baseline: 1.1353x; 1.1353x over previous
import jax
import jax.numpy as jnp
from jax.experimental import pallas as pl
from jax.experimental.pallas import tpu as pltpu

_BR = 256
_BC = 128
_DEPTH = 10
_OD = 8


def _body(nrows_ref, ncols_ref, x_hbm, o_hbm, xbuf, obuf, sems, osems):
    c = o_hbm.shape[2]
    n_rb = x_hbm.shape[1] // _BR
    n_cb = c // _BC
    num_steps = pl.num_programs(0)
    g = pl.program_id(0)
    s = jax.lax.rem(g, _DEPTH)
    so = jax.lax.rem(g, _OD)

    def chunk_copy(gi, slot, k):
        b = gi // n_rb
        rb = jax.lax.rem(gi, n_rb)
        row0 = rb * _BR
        return pltpu.make_async_copy(
            x_hbm.at[b, pl.ds(row0, _BR), pl.ds(k * _BC, _BC)],
            xbuf.at[slot, :, pl.ds(k * _BC, _BC)],
            sems.at[slot, k],
        )

    def live_chunk(gi, k):
        b = gi // n_rb
        rb = jax.lax.rem(gi, n_rb)
        return (rb * _BR < nrows_ref[b]) & (k * _BC < ncols_ref[b])

    def start_copies(gi, slot):
        for k in range(n_cb):
            @pl.when(live_chunk(gi, k))
            def _():
                chunk_copy(gi, slot, k).start()

    def wait_copies(gi, slot):
        for k in range(n_cb):
            @pl.when(live_chunk(gi, k))
            def _():
                chunk_copy(gi, slot, k).wait()

    def out_copy(gi, slot):
        b = gi // n_rb
        rb = jax.lax.rem(gi, n_rb)
        return pltpu.make_async_copy(
            obuf.at[slot],
            o_hbm.at[b, pl.ds(rb * _BR, _BR), :],
            osems.at[slot],
        )

    @pl.when(g == 0)
    def _():
        for i in range(_DEPTH - 1):
            start_copies(i, i)

    @pl.when(g + (_DEPTH - 1) < num_steps)
    def _():
        start_copies(g + (_DEPTH - 1), jax.lax.rem(g + (_DEPTH - 1), _DEPTH))

    @pl.when(g >= _OD)
    def _():
        out_copy(g - _OD, so).wait()

    wait_copies(g, s)

    b = g // n_rb
    rb = jax.lax.rem(g, n_rb)
    nr = nrows_ref[b]
    nc = ncols_ref[b]
    row0 = rb * _BR

    @pl.when(row0 >= nr)
    def _():
        obuf[so] = jnp.zeros((_BR, c), jnp.float32)

    @pl.when(row0 < nr)
    def _():
        x = xbuf[s]
        colmask = jax.lax.broadcasted_iota(jnp.int32, (_BR, c), 1) < nc
        masked = jnp.where(colmask, x, -jnp.inf)
        m = jnp.max(masked, axis=1, keepdims=True)
        safe_m = jnp.where(jnp.isfinite(m), m, 0.0)
        e = jnp.exp(masked - safe_m)
        denom = jnp.sum(e, axis=1, keepdims=True)
        rowvalid = (row0 + jax.lax.broadcasted_iota(jnp.int32, (_BR, 1), 0)) < nr
        inv = jnp.where(rowvalid & (denom > 0),
                        1.0 / jnp.maximum(denom, 1e-30), 0.0)
        obuf[so] = e * inv

    out_copy(g, so).start()

    @pl.when(g == num_steps - 1)
    def _():
        for i in range(_OD):
            out_copy(g - i, jax.lax.rem(g - i, _OD)).wait()


def kernel(sim_mat, nrows, ncols):
    bsz, r, c = sim_mat.shape
    n_rb = r // _BR

    grid_spec = pltpu.PrefetchScalarGridSpec(
        num_scalar_prefetch=2,
        grid=(bsz * n_rb,),
        in_specs=[pl.BlockSpec(memory_space=pltpu.MemorySpace.HBM)],
        out_specs=pl.BlockSpec(memory_space=pltpu.MemorySpace.HBM),
        scratch_shapes=[
            pltpu.VMEM((_DEPTH, _BR, c), jnp.float32),
            pltpu.VMEM((_OD, _BR, c), jnp.float32),
            pltpu.SemaphoreType.DMA((_DEPTH, c // _BC)),
            pltpu.SemaphoreType.DMA((_OD,)),
        ],
    )
    return pl.pallas_call(
        _body,
        grid_spec=grid_spec,
        out_shape=jax.ShapeDtypeStruct((bsz, r, c), sim_mat.dtype),
    )(nrows.astype(jnp.int32), ncols.astype(jnp.int32), sim_mat)
